# baseline (speedup 1.0000x reference)
import functools

import jax
import jax.numpy as jnp
from jax import lax
from jax.experimental import pallas as pl
from jax.experimental.pallas import tpu as pltpu
from jax.experimental.pallas import tpu_sc as plsc

T = 2048
H = 2048
E = 8
F = 512
KTOP = 2

TM = 128
G = T * KTOP // TM + E
P = G * TM

NC, NS, L = 2, 16, 16
NW = NC * NS
TPW = T // NW
RB = 256
WB = 128
H2 = H // 2


def _router_body(x_ref, gw_ref, e1_ref, e2_ref, r1_ref, r2_ref, w1_ref,
                 w2_ref, offs_ref, te_ref, aux_ref, xb_ref, run_ref, ps_ref):
    i = pl.program_id(0)

    @pl.when(i == 0)
    def _():
        run_ref[...] = jnp.zeros_like(run_ref)
        ps_ref[...] = jnp.zeros_like(ps_ref)

    x = x_ref[...]
    xlo = lax.bitcast_convert_type(
        x[:, :H2].astype(jnp.bfloat16), jnp.uint16).astype(jnp.uint32)
    xhi = lax.bitcast_convert_type(
        x[:, H2:].astype(jnp.bfloat16), jnp.uint16).astype(jnp.uint32)
    xb_ref[...] = lax.bitcast_convert_type(xlo | (xhi << 16), jnp.int32)
    logits = lax.dot_general(x, gw_ref[...], (((1,), (1,)), ((), ())),
                             preferred_element_type=jnp.float32)
    m = jnp.max(logits, axis=-1, keepdims=True)
    ex = jnp.exp(logits - m)
    probs = ex / jnp.sum(ex, axis=-1, keepdims=True)

    cols = lax.broadcasted_iota(jnp.int32, (RB, E), 1)
    p1 = jnp.max(probs, axis=-1)
    i1 = jnp.min(jnp.where(probs == p1[:, None], cols, E), axis=-1)
    oh1 = cols == i1[:, None]
    probs2 = jnp.where(oh1, -1.0, probs)
    p2 = jnp.max(probs2, axis=-1)
    i2 = jnp.min(jnp.where(probs2 == p2[:, None], cols, E), axis=-1)
    oh2 = cols == i2[:, None]
    den = p1 + p2 + 1e-9
    w1 = p1 / den
    w2 = p2 / den

    mask = oh1.astype(jnp.float32) + oh2.astype(jnp.float32)
    rr = lax.broadcasted_iota(jnp.int32, (RB, RB), 0)
    cc = lax.broadcasted_iota(jnp.int32, (RB, RB), 1)
    tril = (cc < rr).astype(jnp.float32)
    excl = lax.dot_general(tril, mask, (((1,), (0,)), ((), ())),
                           preferred_element_type=jnp.float32)
    rank = run_ref[...] + excl
    r1 = jnp.sum(rank * oh1, axis=-1)
    r2 = jnp.sum(rank * oh2, axis=-1)

    run_ref[...] = run_ref[...] + jnp.sum(mask, axis=0, keepdims=True)
    ps_ref[...] = ps_ref[...] + jnp.sum(probs, axis=0, keepdims=True)

    e1_ref[0, 0, :] = i1.astype(jnp.int32)
    e2_ref[0, 0, :] = i2.astype(jnp.int32)
    r1_ref[0, 0, :] = r1.astype(jnp.int32)
    r2_ref[0, 0, :] = r2.astype(jnp.int32)
    w1_ref[...] = jnp.broadcast_to(w1[:, None], (RB, WB))
    w2_ref[...] = jnp.broadcast_to(w2[:, None], (RB, WB))

    @pl.when(i == pl.num_programs(0) - 1)
    def _():
        counts = run_ref[...]
        padded = jnp.ceil(counts / TM) * TM
        er = lax.broadcasted_iota(jnp.int32, (E, E), 0)
        ec = lax.broadcasted_iota(jnp.int32, (E, E), 1)
        incl = (er <= ec).astype(jnp.float32)
        ends = lax.dot_general(padded, incl, (((1,), (0,)), ((), ())),
                               preferred_element_type=jnp.float32)
        offs = ends - padded
        offs_ref[0, 0, :] = jnp.concatenate(
            [offs, jnp.zeros((1, 8), jnp.float32)],
            axis=-1).astype(jnp.int32).reshape(16)
        gstart = (lax.broadcasted_iota(jnp.int32, (64, E), 0) * TM
                  ).astype(jnp.float32)
        te = jnp.sum((gstart >= ends).astype(jnp.int32), axis=-1)
        te_ref[0, 0, :] = jnp.minimum(te, E - 1)
        f_i = counts / (T * KTOP)
        p_i = ps_ref[...] / T
        aux_ref[...] = 0.01 * E * jnp.sum(f_i * p_i, keepdims=True)


def _router_call(x, gate_w, interpret=False):
    nb = T // RB
    return pl.pallas_call(
        _router_body,
        grid=(nb,),
        in_specs=[
            pl.BlockSpec((RB, H), lambda i: (i, 0)),
            pl.BlockSpec((E, H), lambda i: (0, 0)),
        ],
        out_specs=[
            pl.BlockSpec((1, 1, RB), lambda i: (i, 0, 0)),
            pl.BlockSpec((1, 1, RB), lambda i: (i, 0, 0)),
            pl.BlockSpec((1, 1, RB), lambda i: (i, 0, 0)),
            pl.BlockSpec((1, 1, RB), lambda i: (i, 0, 0)),
            pl.BlockSpec((RB, WB), lambda i: (i, 0)),
            pl.BlockSpec((RB, WB), lambda i: (i, 0)),
            pl.BlockSpec((1, 1, 16), lambda i: (0, 0, 0)),
            pl.BlockSpec((1, 1, 64), lambda i: (0, 0, 0)),
            pl.BlockSpec((1, 1), lambda i: (0, 0)),
            pl.BlockSpec((RB, H2), lambda i: (i, 0)),
        ],
        out_shape=[
            jax.ShapeDtypeStruct((nb, 1, RB), jnp.int32),
            jax.ShapeDtypeStruct((nb, 1, RB), jnp.int32),
            jax.ShapeDtypeStruct((nb, 1, RB), jnp.int32),
            jax.ShapeDtypeStruct((nb, 1, RB), jnp.int32),
            jax.ShapeDtypeStruct((T, WB), jnp.float32),
            jax.ShapeDtypeStruct((T, WB), jnp.float32),
            jax.ShapeDtypeStruct((1, 1, 16), jnp.int32),
            jax.ShapeDtypeStruct((1, 1, 64), jnp.int32),
            jax.ShapeDtypeStruct((1, 1), jnp.float32),
            jax.ShapeDtypeStruct((T, H2), jnp.int32),
        ],
        scratch_shapes=[
            pltpu.VMEM((1, E), jnp.float32),
            pltpu.VMEM((1, E), jnp.float32),
        ],
        interpret=interpret,
    )(x, gate_w)


def _slot_body(offs_ref, e1_ref, e2_ref, r1_ref, r2_ref, s1_ref, s2_ref):
    e1 = e1_ref[...]
    e2 = e2_ref[...]
    s1 = r1_ref[...]
    s2 = r2_ref[...]
    for e in range(E):
        off_e = offs_ref[e]
        s1 = s1 + jnp.where(e1 == e, off_e, 0)
        s2 = s2 + jnp.where(e2 == e, off_e, 0)
    s1_ref[...] = s1
    s2_ref[...] = s2


def _slot_call(offs, e1, e2, r1, r2, interpret=False):
    nb = T // RB
    grid_spec = pltpu.PrefetchScalarGridSpec(
        num_scalar_prefetch=1,
        grid=(nb,),
        in_specs=[pl.BlockSpec((1, 1, RB), lambda i, offs: (i, 0, 0))] * 4,
        out_specs=[pl.BlockSpec((1, 1, RB), lambda i, offs: (i, 0, 0))] * 2,
    )
    return pl.pallas_call(
        _slot_body,
        grid_spec=grid_spec,
        out_shape=[jax.ShapeDtypeStruct((nb, 1, RB), jnp.int32)] * 2,
        interpret=interpret,
    )(offs, e1, e2, r1, r2)


@functools.cache
def _make_dispatch():
  mesh = plsc.VectorSubcoreMesh(
      core_axis_name="c", subcore_axis_name="s", num_cores=NC, num_subcores=NS)

  @functools.partial(
    pl.kernel,
    out_type=[
        jax.ShapeDtypeStruct((P, H2), jnp.int32),
        jax.ShapeDtypeStruct((P, WB), jnp.float32),
    ],
    mesh=mesh,
    scratch_types=[
        pltpu.VMEM((TPW,), jnp.int32),
        pltpu.VMEM((TPW,), jnp.int32),
        pltpu.VMEM((TPW, WB), jnp.float32),
        pltpu.VMEM((TPW, WB), jnp.float32),
        pltpu.VMEM((L, H2), jnp.int32),
        pltpu.SemaphoreType.DMA,
    ],
  )
  def dispatch(x_hbm, s1_hbm, s2_hbm, w1_hbm, w2_hbm,
               gx_hbm, ws_hbm,
               s1v, s2v, w1v, w2v, rowsv, sem):
    wid = lax.axis_index("s") * NC + lax.axis_index("c")
    base = wid * TPW
    tsl = pl.ds(base, TPW)
    pltpu.sync_copy(s1_hbm.at[tsl], s1v)
    pltpu.sync_copy(s2_hbm.at[tsl], s2v)
    pltpu.sync_copy(w1_hbm.at[tsl], w1v)
    pltpu.sync_copy(w2_hbm.at[tsl], w2v)
    for c in range(TPW // L):
        sl = pl.ds(c * L, L)
        sv1 = s1v[sl]
        sv2 = s2v[sl]
        pltpu.sync_copy(x_hbm.at[pl.ds(base + c * L, L)], rowsv)
        d1 = pltpu.async_copy(rowsv, gx_hbm.at[sv1], sem)
        d2 = pltpu.async_copy(rowsv, gx_hbm.at[sv2], sem)
        d3 = pltpu.async_copy(w1v.at[sl], ws_hbm.at[sv1], sem)
        d4 = pltpu.async_copy(w2v.at[sl], ws_hbm.at[sv2], sem)
        d1.wait()
        d2.wait()
        d3.wait()
        d4.wait()

  return dispatch


def _ffn_body(te_ref, gx_ref, wg_ref, wu_ref, wd_ref, ws_ref, y_ref):
    wrd = lax.bitcast_convert_type(gx_ref[...], jnp.uint32)
    b_lo = lax.bitcast_convert_type(wrd << 16, jnp.float32).astype(jnp.bfloat16)
    b_hi = lax.bitcast_convert_type(
        wrd & jnp.uint32(0xFFFF0000), jnp.float32).astype(jnp.bfloat16)
    wgb = wg_ref[0].astype(jnp.bfloat16)
    wub = wu_ref[0].astype(jnp.bfloat16)
    g = (jnp.dot(b_lo, wgb[:H2], preferred_element_type=jnp.float32)
         + jnp.dot(b_hi, wgb[H2:], preferred_element_type=jnp.float32))
    u = (jnp.dot(b_lo, wub[:H2], preferred_element_type=jnp.float32)
         + jnp.dot(b_hi, wub[H2:], preferred_element_type=jnp.float32))
    act = (g * jax.nn.sigmoid(g) * u).astype(jnp.bfloat16)
    y = jnp.dot(act, wd_ref[0].astype(jnp.bfloat16),
                preferred_element_type=jnp.float32)
    yw = y * ws_ref[:, 0:1]
    ylo = lax.bitcast_convert_type(
        yw[:, :H2].astype(jnp.bfloat16), jnp.uint16).astype(jnp.uint32)
    yhi = lax.bitcast_convert_type(
        yw[:, H2:].astype(jnp.bfloat16), jnp.uint16).astype(jnp.uint32)
    y_ref[...] = lax.bitcast_convert_type(ylo | (yhi << 16), jnp.int32)


def _ffn_call(tile_e, gx, w_gate, w_up, w_down, wslot, interpret=False):
    grid_spec = pltpu.PrefetchScalarGridSpec(
        num_scalar_prefetch=1,
        grid=(G,),
        in_specs=[
            pl.BlockSpec((TM, H2), lambda g, te: (g, 0)),
            pl.BlockSpec((1, H, F), lambda g, te: (te[g], 0, 0)),
            pl.BlockSpec((1, H, F), lambda g, te: (te[g], 0, 0)),
            pl.BlockSpec((1, F, H), lambda g, te: (te[g], 0, 0)),
            pl.BlockSpec((TM, WB), lambda g, te: (g, 0)),
        ],
        out_specs=pl.BlockSpec((TM, H2), lambda g, te: (g, 0)),
    )
    return pl.pallas_call(
        _ffn_body,
        grid_spec=grid_spec,
        out_shape=jax.ShapeDtypeStruct((P, H2), jnp.int32),
        interpret=interpret,
    )(tile_e, gx, w_gate, w_up, w_down, wslot)


@functools.cache
def _make_combine():
  mesh = plsc.VectorSubcoreMesh(
      core_axis_name="c", subcore_axis_name="s", num_cores=NC, num_subcores=NS)

  CS = 8
  NCH = TPW // CS

  @functools.partial(
    pl.kernel,
    out_type=jax.ShapeDtypeStruct((T, H), jnp.float32),
    mesh=mesh,
    scratch_types=[
        pltpu.VMEM((TPW,), jnp.int32),
        pltpu.VMEM((TPW,), jnp.int32),
        pltpu.VMEM((2, CS, H2), jnp.int32),
        pltpu.VMEM((2, CS, H2), jnp.int32),
        pltpu.VMEM((2, CS, H), jnp.float32),
        pltpu.SemaphoreType.DMA,
        pltpu.SemaphoreType.DMA,
        pltpu.SemaphoreType.DMA,
        pltpu.SemaphoreType.DMA,
    ],
  )
  def combine(y_hbm, s1_hbm, s2_hbm, out_hbm,
              s1v, s2v, b1, b2, fout, sg0, sg1, so0, so1):
    wid = lax.axis_index("s") * NC + lax.axis_index("c")
    base = wid * TPW
    tsl = pl.ds(base, TPW)
    pltpu.sync_copy(s1_hbm.at[tsl], s1v)
    pltpu.sync_copy(s2_hbm.at[tsl], s2v)
    sg = (sg0, sg1)
    so = (so0, so1)

    def fire(c):
        par = c % 2
        isl = pl.ds(c * CS, CS)
        d1 = pltpu.async_copy(y_hbm.at[s1v.at[isl]], b1.at[par], sg[par])
        d2 = pltpu.async_copy(y_hbm.at[s2v.at[isl]], b2.at[par], sg[par])
        return d1, d2

    descs = [None] * NCH
    outd = [None] * NCH
    descs[0] = fire(0)
    for c in range(NCH):
        par = c % 2
        if c + 1 < NCH:
            if c >= 1:
                outd[c - 1].wait()
            descs[c + 1] = fire(c + 1)
        d1, d2 = descs[c]
        d1.wait()
        d2.wait()
        for i in range(CS):
            @plsc.parallel_loop(0, H2 // L, 1, unroll=4)
            def _(j, i=i, par=par):
                sl2 = pl.ds(j * L, L)
                wa = lax.bitcast_convert_type(b1[par, i, sl2], jnp.uint32)
                wb = lax.bitcast_convert_type(b2[par, i, sl2], jnp.uint32)
                fout[par, i, sl2] = (
                    lax.bitcast_convert_type(wa << 16, jnp.float32)
                    + lax.bitcast_convert_type(wb << 16, jnp.float32))
                fout[par, i, pl.ds(H2 + j * L, L)] = (
                    lax.bitcast_convert_type(
                        wa & jnp.uint32(0xFFFF0000), jnp.float32)
                    + lax.bitcast_convert_type(
                        wb & jnp.uint32(0xFFFF0000), jnp.float32))
        outd[c] = pltpu.async_copy(
            fout.at[par], out_hbm.at[pl.ds(base + c * CS, CS)], so[par])
    outd[NCH - 2].wait()
    outd[NCH - 1].wait()

  return combine


def kernel(hidden_states, gate_w, w_gate, w_up, w_down):
    b, s, h = hidden_states.shape
    x = hidden_states.reshape(-1, h)

    (e1, e2, r1, r2, w1b, w2b, offs, te, aux, xb) = _router_call(x, gate_w)

    s1, s2 = _slot_call(offs.reshape(16), e1, e2, r1, r2)
    s1 = s1.reshape(T)
    s2 = s2.reshape(T)

    gx, wslot = _make_dispatch()(xb, s1, s2, w1b, w2b)

    y = _ffn_call(te.reshape(64)[:G], gx, w_gate, w_up, w_down, wslot)

    out = _make_combine()(y, s1, s2)
    return out.reshape(b, s, h), aux.reshape(())


# speedup vs baseline: 1.0048x; 1.0048x over previous
import functools

import jax
import jax.numpy as jnp
from jax import lax
from jax.experimental import pallas as pl
from jax.experimental.pallas import tpu as pltpu
from jax.experimental.pallas import tpu_sc as plsc

T = 2048
H = 2048
E = 8
F = 512
KTOP = 2

TM = 128
G = T * KTOP // TM + E
P = G * TM

NC, NS, L = 2, 16, 16
NW = NC * NS
TPW = T // NW
RB = 256
WB = 128
H2 = H // 2


def _router_body(x_ref, gw_ref, e1_ref, e2_ref, r1_ref, r2_ref, w1_ref,
                 w2_ref, offs_ref, te_ref, aux_ref, xb_ref, run_ref, ps_ref):
    i = pl.program_id(0)

    @pl.when(i == 0)
    def _():
        run_ref[...] = jnp.zeros_like(run_ref)
        ps_ref[...] = jnp.zeros_like(ps_ref)

    x = x_ref[...]
    xlo = lax.bitcast_convert_type(
        x[:, :H2].astype(jnp.bfloat16), jnp.uint16).astype(jnp.uint32)
    xhi = lax.bitcast_convert_type(
        x[:, H2:].astype(jnp.bfloat16), jnp.uint16).astype(jnp.uint32)
    xb_ref[...] = lax.bitcast_convert_type(xlo | (xhi << 16), jnp.int32)
    logits = lax.dot_general(x, gw_ref[...], (((1,), (1,)), ((), ())),
                             preferred_element_type=jnp.float32)
    m = jnp.max(logits, axis=-1, keepdims=True)
    ex = jnp.exp(logits - m)
    probs = ex / jnp.sum(ex, axis=-1, keepdims=True)

    cols = lax.broadcasted_iota(jnp.int32, (RB, E), 1)
    p1 = jnp.max(probs, axis=-1)
    i1 = jnp.min(jnp.where(probs == p1[:, None], cols, E), axis=-1)
    oh1 = cols == i1[:, None]
    probs2 = jnp.where(oh1, -1.0, probs)
    p2 = jnp.max(probs2, axis=-1)
    i2 = jnp.min(jnp.where(probs2 == p2[:, None], cols, E), axis=-1)
    oh2 = cols == i2[:, None]
    den = p1 + p2 + 1e-9
    w1 = p1 / den
    w2 = p2 / den

    mask = oh1.astype(jnp.float32) + oh2.astype(jnp.float32)
    rr = lax.broadcasted_iota(jnp.int32, (RB, RB), 0)
    cc = lax.broadcasted_iota(jnp.int32, (RB, RB), 1)
    tril = (cc < rr).astype(jnp.float32)
    excl = lax.dot_general(tril, mask, (((1,), (0,)), ((), ())),
                           preferred_element_type=jnp.float32)
    rank = run_ref[...] + excl
    r1 = jnp.sum(rank * oh1, axis=-1)
    r2 = jnp.sum(rank * oh2, axis=-1)

    run_ref[...] = run_ref[...] + jnp.sum(mask, axis=0, keepdims=True)
    ps_ref[...] = ps_ref[...] + jnp.sum(probs, axis=0, keepdims=True)

    e1_ref[0, 0, :] = i1.astype(jnp.int32)
    e2_ref[0, 0, :] = i2.astype(jnp.int32)
    r1_ref[0, 0, :] = r1.astype(jnp.int32)
    r2_ref[0, 0, :] = r2.astype(jnp.int32)
    w1_ref[...] = jnp.broadcast_to(w1[:, None], (RB, WB))
    w2_ref[...] = jnp.broadcast_to(w2[:, None], (RB, WB))

    @pl.when(i == pl.num_programs(0) - 1)
    def _():
        counts = run_ref[...]
        padded = jnp.ceil(counts / TM) * TM
        er = lax.broadcasted_iota(jnp.int32, (E, E), 0)
        ec = lax.broadcasted_iota(jnp.int32, (E, E), 1)
        incl = (er <= ec).astype(jnp.float32)
        ends = lax.dot_general(padded, incl, (((1,), (0,)), ((), ())),
                               preferred_element_type=jnp.float32)
        offs = ends - padded
        offs_ref[0, 0, :] = jnp.concatenate(
            [offs, jnp.zeros((1, 8), jnp.float32)],
            axis=-1).astype(jnp.int32).reshape(16)
        gstart = (lax.broadcasted_iota(jnp.int32, (64, E), 0) * TM
                  ).astype(jnp.float32)
        te = jnp.sum((gstart >= ends).astype(jnp.int32), axis=-1)
        te_ref[0, 0, :] = jnp.minimum(te, E - 1)
        f_i = counts / (T * KTOP)
        p_i = ps_ref[...] / T
        aux_ref[...] = 0.01 * E * jnp.sum(f_i * p_i, keepdims=True)


def _router_call(x, gate_w):
    nb = T // RB
    return pl.pallas_call(
        _router_body,
        grid=(nb,),
        in_specs=[
            pl.BlockSpec((RB, H), lambda i: (i, 0)),
            pl.BlockSpec((E, H), lambda i: (0, 0)),
        ],
        out_specs=[
            pl.BlockSpec((1, 1, RB), lambda i: (i, 0, 0)),
            pl.BlockSpec((1, 1, RB), lambda i: (i, 0, 0)),
            pl.BlockSpec((1, 1, RB), lambda i: (i, 0, 0)),
            pl.BlockSpec((1, 1, RB), lambda i: (i, 0, 0)),
            pl.BlockSpec((RB, WB), lambda i: (i, 0)),
            pl.BlockSpec((RB, WB), lambda i: (i, 0)),
            pl.BlockSpec((1, 1, 16), lambda i: (0, 0, 0)),
            pl.BlockSpec((1, 1, 64), lambda i: (0, 0, 0)),
            pl.BlockSpec((1, 1), lambda i: (0, 0)),
            pl.BlockSpec((RB, H2), lambda i: (i, 0)),
        ],
        out_shape=[
            jax.ShapeDtypeStruct((nb, 1, RB), jnp.int32),
            jax.ShapeDtypeStruct((nb, 1, RB), jnp.int32),
            jax.ShapeDtypeStruct((nb, 1, RB), jnp.int32),
            jax.ShapeDtypeStruct((nb, 1, RB), jnp.int32),
            jax.ShapeDtypeStruct((T, WB), jnp.float32),
            jax.ShapeDtypeStruct((T, WB), jnp.float32),
            jax.ShapeDtypeStruct((1, 1, 16), jnp.int32),
            jax.ShapeDtypeStruct((1, 1, 64), jnp.int32),
            jax.ShapeDtypeStruct((1, 1), jnp.float32),
            jax.ShapeDtypeStruct((T, H2), jnp.int32),
        ],
        scratch_shapes=[
            pltpu.VMEM((1, E), jnp.float32),
            pltpu.VMEM((1, E), jnp.float32),
        ],
    )(x, gate_w)


def _slot_body(offs_ref, e1_ref, e2_ref, r1_ref, r2_ref, s1_ref, s2_ref):
    e1 = e1_ref[...]
    e2 = e2_ref[...]
    s1 = r1_ref[...]
    s2 = r2_ref[...]
    for e in range(E):
        off_e = offs_ref[e]
        s1 = s1 + jnp.where(e1 == e, off_e, 0)
        s2 = s2 + jnp.where(e2 == e, off_e, 0)
    s1_ref[...] = s1
    s2_ref[...] = s2


def _slot_call(offs, e1, e2, r1, r2):
    nb = T // RB
    grid_spec = pltpu.PrefetchScalarGridSpec(
        num_scalar_prefetch=1,
        grid=(nb,),
        in_specs=[pl.BlockSpec((1, 1, RB), lambda i, offs: (i, 0, 0))] * 4,
        out_specs=[pl.BlockSpec((1, 1, RB), lambda i, offs: (i, 0, 0))] * 2,
    )
    return pl.pallas_call(
        _slot_body,
        grid_spec=grid_spec,
        out_shape=[jax.ShapeDtypeStruct((nb, 1, RB), jnp.int32)] * 2,
    )(offs, e1, e2, r1, r2)


@functools.cache
def _make_dispatch():
  mesh = plsc.VectorSubcoreMesh(
      core_axis_name="c", subcore_axis_name="s", num_cores=NC, num_subcores=NS)

  @functools.partial(
    pl.kernel,
    out_type=[
        jax.ShapeDtypeStruct((P, H2), jnp.int32),
        jax.ShapeDtypeStruct((P, WB), jnp.float32),
    ],
    mesh=mesh,
    scratch_types=[
        pltpu.VMEM((TPW,), jnp.int32),
        pltpu.VMEM((TPW,), jnp.int32),
        pltpu.VMEM((TPW, WB), jnp.float32),
        pltpu.VMEM((TPW, WB), jnp.float32),
        pltpu.VMEM((L, H2), jnp.int32),
        pltpu.SemaphoreType.DMA,
    ],
  )
  def dispatch(x_hbm, s1_hbm, s2_hbm, w1_hbm, w2_hbm,
               gx_hbm, ws_hbm,
               s1v, s2v, w1v, w2v, rowsv, sem):
    wid = lax.axis_index("s") * NC + lax.axis_index("c")
    base = wid * TPW
    tsl = pl.ds(base, TPW)
    pltpu.sync_copy(s1_hbm.at[tsl], s1v)
    pltpu.sync_copy(s2_hbm.at[tsl], s2v)
    pltpu.sync_copy(w1_hbm.at[tsl], w1v)
    pltpu.sync_copy(w2_hbm.at[tsl], w2v)
    for c in range(TPW // L):
        sl = pl.ds(c * L, L)
        sv1 = s1v[sl]
        sv2 = s2v[sl]
        pltpu.sync_copy(x_hbm.at[pl.ds(base + c * L, L)], rowsv)
        d1 = pltpu.async_copy(rowsv, gx_hbm.at[sv1], sem)
        d2 = pltpu.async_copy(rowsv, gx_hbm.at[sv2], sem)
        d3 = pltpu.async_copy(w1v.at[sl], ws_hbm.at[sv1], sem)
        d4 = pltpu.async_copy(w2v.at[sl], ws_hbm.at[sv2], sem)
        d1.wait()
        d2.wait()
        d3.wait()
        d4.wait()

  return dispatch


def _ffn_body(te_ref, gx_ref, wg_ref, wu_ref, wd_ref, ws_ref, y_ref):
    wrd = lax.bitcast_convert_type(gx_ref[...], jnp.uint32)
    f_lo = lax.bitcast_convert_type(wrd << 16, jnp.float32)
    f_hi = lax.bitcast_convert_type(wrd & jnp.uint32(0xFFFF0000), jnp.float32)
    g = (jnp.dot(f_lo, wg_ref[0, :H2], preferred_element_type=jnp.float32)
         + jnp.dot(f_hi, wg_ref[0, H2:], preferred_element_type=jnp.float32))
    u = (jnp.dot(f_lo, wu_ref[0, :H2], preferred_element_type=jnp.float32)
         + jnp.dot(f_hi, wu_ref[0, H2:], preferred_element_type=jnp.float32))
    act = g * jax.nn.sigmoid(g) * u
    y = jnp.dot(act, wd_ref[0], preferred_element_type=jnp.float32)
    yw = y * ws_ref[:, 0:1]
    ylo = lax.bitcast_convert_type(
        yw[:, :H2].astype(jnp.bfloat16), jnp.uint16).astype(jnp.uint32)
    yhi = lax.bitcast_convert_type(
        yw[:, H2:].astype(jnp.bfloat16), jnp.uint16).astype(jnp.uint32)
    y_ref[...] = lax.bitcast_convert_type(ylo | (yhi << 16), jnp.int32)


def _ffn_call(tile_e, gx, w_gate, w_up, w_down, wslot):
    grid_spec = pltpu.PrefetchScalarGridSpec(
        num_scalar_prefetch=1,
        grid=(G,),
        in_specs=[
            pl.BlockSpec((TM, H2), lambda g, te: (g, 0)),
            pl.BlockSpec((1, H, F), lambda g, te: (te[g], 0, 0)),
            pl.BlockSpec((1, H, F), lambda g, te: (te[g], 0, 0)),
            pl.BlockSpec((1, F, H), lambda g, te: (te[g], 0, 0)),
            pl.BlockSpec((TM, WB), lambda g, te: (g, 0)),
        ],
        out_specs=pl.BlockSpec((TM, H2), lambda g, te: (g, 0)),
    )
    return pl.pallas_call(
        _ffn_body,
        grid_spec=grid_spec,
        out_shape=jax.ShapeDtypeStruct((P, H2), jnp.int32),
    )(tile_e, gx, w_gate, w_up, w_down, wslot)


@functools.cache
def _make_combine():
  mesh = plsc.VectorSubcoreMesh(
      core_axis_name="c", subcore_axis_name="s", num_cores=NC, num_subcores=NS)

  CS = 8
  NCH = TPW // CS

  @functools.partial(
    pl.kernel,
    out_type=jax.ShapeDtypeStruct((T, H), jnp.float32),
    mesh=mesh,
    scratch_types=[
        pltpu.VMEM((TPW,), jnp.int32),
        pltpu.VMEM((TPW,), jnp.int32),
        pltpu.VMEM((2, CS, H2), jnp.int32),
        pltpu.VMEM((2, CS, H2), jnp.int32),
        pltpu.VMEM((2, CS, H), jnp.float32),
        pltpu.SemaphoreType.DMA,
        pltpu.SemaphoreType.DMA,
        pltpu.SemaphoreType.DMA,
        pltpu.SemaphoreType.DMA,
    ],
  )
  def combine(y_hbm, s1_hbm, s2_hbm, out_hbm,
              s1v, s2v, b1, b2, fout, sg0, sg1, so0, so1):
    wid = lax.axis_index("s") * NC + lax.axis_index("c")
    base = wid * TPW
    tsl = pl.ds(base, TPW)
    pltpu.sync_copy(s1_hbm.at[tsl], s1v)
    pltpu.sync_copy(s2_hbm.at[tsl], s2v)
    sg = (sg0, sg1)
    so = (so0, so1)

    def fire(c):
        par = c % 2
        isl = pl.ds(c * CS, CS)
        d1 = pltpu.async_copy(y_hbm.at[s1v.at[isl]], b1.at[par], sg[par])
        d2 = pltpu.async_copy(y_hbm.at[s2v.at[isl]], b2.at[par], sg[par])
        return d1, d2

    descs = [None] * NCH
    outd = [None] * NCH
    descs[0] = fire(0)
    for c in range(NCH):
        par = c % 2
        if c + 1 < NCH:
            if c >= 1:
                outd[c - 1].wait()
            descs[c + 1] = fire(c + 1)
        d1, d2 = descs[c]
        d1.wait()
        d2.wait()
        for i in range(CS):
            @plsc.parallel_loop(0, H2 // L, 1, unroll=4)
            def _(j, i=i, par=par):
                sl2 = pl.ds(j * L, L)
                wa = lax.bitcast_convert_type(b1[par, i, sl2], jnp.uint32)
                wb = lax.bitcast_convert_type(b2[par, i, sl2], jnp.uint32)
                fout[par, i, sl2] = (
                    lax.bitcast_convert_type(wa << 16, jnp.float32)
                    + lax.bitcast_convert_type(wb << 16, jnp.float32))
                fout[par, i, pl.ds(H2 + j * L, L)] = (
                    lax.bitcast_convert_type(
                        wa & jnp.uint32(0xFFFF0000), jnp.float32)
                    + lax.bitcast_convert_type(
                        wb & jnp.uint32(0xFFFF0000), jnp.float32))
        outd[c] = pltpu.async_copy(
            fout.at[par], out_hbm.at[pl.ds(base + c * CS, CS)], so[par])
    outd[NCH - 2].wait()
    outd[NCH - 1].wait()

  return combine


def kernel(hidden_states, gate_w, w_gate, w_up, w_down):
    b, s, h = hidden_states.shape
    x = hidden_states.reshape(-1, h)

    (e1, e2, r1, r2, w1b, w2b, offs, te, aux, xb) = _router_call(x, gate_w)

    s1, s2 = _slot_call(offs.reshape(16), e1, e2, r1, r2)
    s1 = s1.reshape(T)
    s2 = s2.reshape(T)

    gx, wslot = _make_dispatch()(xb, s1, s2, w1b, w2b)

    y = _ffn_call(te.reshape(64)[:G], gx, w_gate, w_up, w_down, wslot)

    out = _make_combine()(y, s1, s2)
    return out.reshape(b, s, h), aux.reshape(())


# speedup vs baseline: 1.0170x; 1.0121x over previous
import functools

import jax
import jax.numpy as jnp
from jax import lax
from jax.experimental import pallas as pl
from jax.experimental.pallas import tpu as pltpu
from jax.experimental.pallas import tpu_sc as plsc

T = 2048
H = 2048
E = 8
F = 512
KTOP = 2

TM = 128
G = T * KTOP // TM + E
P = G * TM

NC, NS, L = 2, 16, 16
NW = NC * NS
TPW = T // NW
RB = 256
WB = 128
H2 = H // 2


def _router_body(x_ref, gw_ref, e1_ref, e2_ref, r1_ref, r2_ref, w1_ref,
                 w2_ref, offs_ref, te_ref, aux_ref, xb_ref, run_ref, ps_ref):
    i = pl.program_id(0)

    @pl.when(i == 0)
    def _():
        run_ref[...] = jnp.zeros_like(run_ref)
        ps_ref[...] = jnp.zeros_like(ps_ref)

    x = x_ref[...]
    xlo = lax.bitcast_convert_type(
        x[:, :H2].astype(jnp.bfloat16), jnp.uint16).astype(jnp.uint32)
    xhi = lax.bitcast_convert_type(
        x[:, H2:].astype(jnp.bfloat16), jnp.uint16).astype(jnp.uint32)
    xb_ref[...] = lax.bitcast_convert_type(xlo | (xhi << 16), jnp.int32)
    logits = lax.dot_general(x, gw_ref[...], (((1,), (1,)), ((), ())),
                             preferred_element_type=jnp.float32)
    m = jnp.max(logits, axis=-1, keepdims=True)
    ex = jnp.exp(logits - m)
    probs = ex / jnp.sum(ex, axis=-1, keepdims=True)

    cols = lax.broadcasted_iota(jnp.int32, (RB, E), 1)
    p1 = jnp.max(probs, axis=-1)
    i1 = jnp.min(jnp.where(probs == p1[:, None], cols, E), axis=-1)
    oh1 = cols == i1[:, None]
    probs2 = jnp.where(oh1, -1.0, probs)
    p2 = jnp.max(probs2, axis=-1)
    i2 = jnp.min(jnp.where(probs2 == p2[:, None], cols, E), axis=-1)
    oh2 = cols == i2[:, None]
    den = p1 + p2 + 1e-9
    w1 = p1 / den
    w2 = p2 / den

    mask = oh1.astype(jnp.float32) + oh2.astype(jnp.float32)
    rr = lax.broadcasted_iota(jnp.int32, (RB, RB), 0)
    cc = lax.broadcasted_iota(jnp.int32, (RB, RB), 1)
    tril = (cc < rr).astype(jnp.float32)
    excl = lax.dot_general(tril, mask, (((1,), (0,)), ((), ())),
                           preferred_element_type=jnp.float32)
    rank = run_ref[...] + excl
    r1 = jnp.sum(rank * oh1, axis=-1)
    r2 = jnp.sum(rank * oh2, axis=-1)

    run_ref[...] = run_ref[...] + jnp.sum(mask, axis=0, keepdims=True)
    ps_ref[...] = ps_ref[...] + jnp.sum(probs, axis=0, keepdims=True)

    e1_ref[0, 0, :] = i1.astype(jnp.int32)
    e2_ref[0, 0, :] = i2.astype(jnp.int32)
    r1_ref[0, 0, :] = r1.astype(jnp.int32)
    r2_ref[0, 0, :] = r2.astype(jnp.int32)
    w1_ref[...] = jnp.broadcast_to(w1[:, None], (RB, WB))
    w2_ref[...] = jnp.broadcast_to(w2[:, None], (RB, WB))

    @pl.when(i == pl.num_programs(0) - 1)
    def _():
        counts = run_ref[...]
        padded = jnp.ceil(counts / TM) * TM
        er = lax.broadcasted_iota(jnp.int32, (E, E), 0)
        ec = lax.broadcasted_iota(jnp.int32, (E, E), 1)
        incl = (er <= ec).astype(jnp.float32)
        ends = lax.dot_general(padded, incl, (((1,), (0,)), ((), ())),
                               preferred_element_type=jnp.float32)
        offs = ends - padded
        offs_ref[0, 0, :] = jnp.concatenate(
            [offs, jnp.zeros((1, 8), jnp.float32)],
            axis=-1).astype(jnp.int32).reshape(16)
        gstart = (lax.broadcasted_iota(jnp.int32, (64, E), 0) * TM
                  ).astype(jnp.float32)
        te = jnp.sum((gstart >= ends).astype(jnp.int32), axis=-1)
        te_ref[0, 0, :] = jnp.minimum(te, E - 1)
        f_i = counts / (T * KTOP)
        p_i = ps_ref[...] / T
        aux_ref[...] = 0.01 * E * jnp.sum(f_i * p_i, keepdims=True)


def _router_call(x, gate_w):
    nb = T // RB
    return pl.pallas_call(
        _router_body,
        grid=(nb,),
        in_specs=[
            pl.BlockSpec((RB, H), lambda i: (i, 0)),
            pl.BlockSpec((E, H), lambda i: (0, 0)),
        ],
        out_specs=[
            pl.BlockSpec((1, 1, RB), lambda i: (i, 0, 0)),
            pl.BlockSpec((1, 1, RB), lambda i: (i, 0, 0)),
            pl.BlockSpec((1, 1, RB), lambda i: (i, 0, 0)),
            pl.BlockSpec((1, 1, RB), lambda i: (i, 0, 0)),
            pl.BlockSpec((RB, WB), lambda i: (i, 0)),
            pl.BlockSpec((RB, WB), lambda i: (i, 0)),
            pl.BlockSpec((1, 1, 16), lambda i: (0, 0, 0)),
            pl.BlockSpec((1, 1, 64), lambda i: (0, 0, 0)),
            pl.BlockSpec((1, 1), lambda i: (0, 0)),
            pl.BlockSpec((RB, H2), lambda i: (i, 0)),
        ],
        out_shape=[
            jax.ShapeDtypeStruct((nb, 1, RB), jnp.int32),
            jax.ShapeDtypeStruct((nb, 1, RB), jnp.int32),
            jax.ShapeDtypeStruct((nb, 1, RB), jnp.int32),
            jax.ShapeDtypeStruct((nb, 1, RB), jnp.int32),
            jax.ShapeDtypeStruct((T, WB), jnp.float32),
            jax.ShapeDtypeStruct((T, WB), jnp.float32),
            jax.ShapeDtypeStruct((1, 1, 16), jnp.int32),
            jax.ShapeDtypeStruct((1, 1, 64), jnp.int32),
            jax.ShapeDtypeStruct((1, 1), jnp.float32),
            jax.ShapeDtypeStruct((T, H2), jnp.int32),
        ],
        scratch_shapes=[
            pltpu.VMEM((1, E), jnp.float32),
            pltpu.VMEM((1, E), jnp.float32),
        ],
    )(x, gate_w)


def _slot_body(offs_ref, e1_ref, e2_ref, r1_ref, r2_ref, s1_ref, s2_ref):
    e1 = e1_ref[...]
    e2 = e2_ref[...]
    s1 = r1_ref[...]
    s2 = r2_ref[...]
    for e in range(E):
        off_e = offs_ref[e]
        s1 = s1 + jnp.where(e1 == e, off_e, 0)
        s2 = s2 + jnp.where(e2 == e, off_e, 0)
    s1_ref[...] = s1
    s2_ref[...] = s2


def _slot_call(offs, e1, e2, r1, r2):
    nb = T // RB
    grid_spec = pltpu.PrefetchScalarGridSpec(
        num_scalar_prefetch=1,
        grid=(nb,),
        in_specs=[pl.BlockSpec((1, 1, RB), lambda i, offs: (i, 0, 0))] * 4,
        out_specs=[pl.BlockSpec((1, 1, RB), lambda i, offs: (i, 0, 0))] * 2,
    )
    return pl.pallas_call(
        _slot_body,
        grid_spec=grid_spec,
        out_shape=[jax.ShapeDtypeStruct((nb, 1, RB), jnp.int32)] * 2,
    )(offs, e1, e2, r1, r2)


@functools.cache
def _make_dispatch():
  mesh = plsc.VectorSubcoreMesh(
      core_axis_name="c", subcore_axis_name="s", num_cores=NC, num_subcores=NS)

  @functools.partial(
    pl.kernel,
    out_type=[
        jax.ShapeDtypeStruct((P, H2), jnp.int32),
        jax.ShapeDtypeStruct((P, WB), jnp.float32),
    ],
    mesh=mesh,
    scratch_types=[
        pltpu.VMEM((TPW,), jnp.int32),
        pltpu.VMEM((TPW,), jnp.int32),
        pltpu.VMEM((TPW, WB), jnp.float32),
        pltpu.VMEM((TPW, WB), jnp.float32),
        pltpu.VMEM((2, L, H2), jnp.int32),
        pltpu.SemaphoreType.DMA,
        pltpu.SemaphoreType.DMA,
        pltpu.SemaphoreType.DMA,
    ],
  )
  def dispatch(x_hbm, s1_hbm, s2_hbm, w1_hbm, w2_hbm,
               gx_hbm, ws_hbm,
               s1v, s2v, w1v, w2v, rowsv, si0, si1, sem):
    wid = lax.axis_index("s") * NC + lax.axis_index("c")
    base = wid * TPW
    tsl = pl.ds(base, TPW)
    pltpu.sync_copy(s1_hbm.at[tsl], s1v)
    pltpu.sync_copy(s2_hbm.at[tsl], s2v)
    pltpu.sync_copy(w1_hbm.at[tsl], w1v)
    pltpu.sync_copy(w2_hbm.at[tsl], w2v)
    si = (si0, si1)
    NCH = TPW // L

    def stage(c):
        return pltpu.async_copy(
            x_hbm.at[pl.ds(base + c * L, L)], rowsv.at[c % 2], si[c % 2])

    ind = [None] * NCH
    scat = [None] * NCH
    ind[0] = stage(0)
    for c in range(NCH):
        par = c % 2
        sl = pl.ds(c * L, L)
        sv1 = s1v[sl]
        sv2 = s2v[sl]
        ind[c].wait()
        d1 = pltpu.async_copy(rowsv.at[par], gx_hbm.at[sv1], sem)
        d2 = pltpu.async_copy(rowsv.at[par], gx_hbm.at[sv2], sem)
        d3 = pltpu.async_copy(w1v.at[sl], ws_hbm.at[sv1], sem)
        d4 = pltpu.async_copy(w2v.at[sl], ws_hbm.at[sv2], sem)
        scat[c] = (d1, d2, d3, d4)
        if c >= 1:
            for d in scat[c - 1]:
                d.wait()
        if c + 1 < NCH:
            ind[c + 1] = stage(c + 1)
    for d in scat[NCH - 1]:
        d.wait()

  return dispatch


def _ffn_body(te_ref, gx_ref, wg_ref, wu_ref, wd_ref, ws_ref, y_ref):
    wrd = lax.bitcast_convert_type(gx_ref[...], jnp.uint32)
    f_lo = lax.bitcast_convert_type(wrd << 16, jnp.float32)
    f_hi = lax.bitcast_convert_type(wrd & jnp.uint32(0xFFFF0000), jnp.float32)
    g = (jnp.dot(f_lo, wg_ref[0, :H2], preferred_element_type=jnp.float32)
         + jnp.dot(f_hi, wg_ref[0, H2:], preferred_element_type=jnp.float32))
    u = (jnp.dot(f_lo, wu_ref[0, :H2], preferred_element_type=jnp.float32)
         + jnp.dot(f_hi, wu_ref[0, H2:], preferred_element_type=jnp.float32))
    act = g * jax.nn.sigmoid(g) * u
    y = jnp.dot(act, wd_ref[0], preferred_element_type=jnp.float32)
    yw = y * ws_ref[:, 0:1]
    ylo = lax.bitcast_convert_type(
        yw[:, :H2].astype(jnp.bfloat16), jnp.uint16).astype(jnp.uint32)
    yhi = lax.bitcast_convert_type(
        yw[:, H2:].astype(jnp.bfloat16), jnp.uint16).astype(jnp.uint32)
    y_ref[...] = lax.bitcast_convert_type(ylo | (yhi << 16), jnp.int32)


def _ffn_call(tile_e, gx, w_gate, w_up, w_down, wslot):
    grid_spec = pltpu.PrefetchScalarGridSpec(
        num_scalar_prefetch=1,
        grid=(G,),
        in_specs=[
            pl.BlockSpec((TM, H2), lambda g, te: (g, 0)),
            pl.BlockSpec((1, H, F), lambda g, te: (te[g], 0, 0)),
            pl.BlockSpec((1, H, F), lambda g, te: (te[g], 0, 0)),
            pl.BlockSpec((1, F, H), lambda g, te: (te[g], 0, 0)),
            pl.BlockSpec((TM, WB), lambda g, te: (g, 0)),
        ],
        out_specs=pl.BlockSpec((TM, H2), lambda g, te: (g, 0)),
    )
    return pl.pallas_call(
        _ffn_body,
        grid_spec=grid_spec,
        out_shape=jax.ShapeDtypeStruct((P, H2), jnp.int32),
    )(tile_e, gx, w_gate, w_up, w_down, wslot)


@functools.cache
def _make_combine():
  mesh = plsc.VectorSubcoreMesh(
      core_axis_name="c", subcore_axis_name="s", num_cores=NC, num_subcores=NS)

  CS = 8
  NCH = TPW // CS

  @functools.partial(
    pl.kernel,
    out_type=jax.ShapeDtypeStruct((T, H), jnp.float32),
    mesh=mesh,
    scratch_types=[
        pltpu.VMEM((TPW,), jnp.int32),
        pltpu.VMEM((TPW,), jnp.int32),
        pltpu.VMEM((2, CS, H2), jnp.int32),
        pltpu.VMEM((2, CS, H2), jnp.int32),
        pltpu.VMEM((2, CS, H), jnp.float32),
        pltpu.SemaphoreType.DMA,
        pltpu.SemaphoreType.DMA,
        pltpu.SemaphoreType.DMA,
        pltpu.SemaphoreType.DMA,
    ],
  )
  def combine(y_hbm, s1_hbm, s2_hbm, out_hbm,
              s1v, s2v, b1, b2, fout, sg0, sg1, so0, so1):
    wid = lax.axis_index("s") * NC + lax.axis_index("c")
    base = wid * TPW
    tsl = pl.ds(base, TPW)
    pltpu.sync_copy(s1_hbm.at[tsl], s1v)
    pltpu.sync_copy(s2_hbm.at[tsl], s2v)
    sg = (sg0, sg1)
    so = (so0, so1)

    def fire(c):
        par = c % 2
        isl = pl.ds(c * CS, CS)
        d1 = pltpu.async_copy(y_hbm.at[s1v.at[isl]], b1.at[par], sg[par])
        d2 = pltpu.async_copy(y_hbm.at[s2v.at[isl]], b2.at[par], sg[par])
        return d1, d2

    descs = [None] * NCH
    outd = [None] * NCH
    descs[0] = fire(0)
    for c in range(NCH):
        par = c % 2
        if c + 1 < NCH:
            if c >= 1:
                outd[c - 1].wait()
            descs[c + 1] = fire(c + 1)
        d1, d2 = descs[c]
        d1.wait()
        d2.wait()
        for i in range(CS):
            @plsc.parallel_loop(0, H2 // L, 1, unroll=4)
            def _(j, i=i, par=par):
                sl2 = pl.ds(j * L, L)
                wa = lax.bitcast_convert_type(b1[par, i, sl2], jnp.uint32)
                wb = lax.bitcast_convert_type(b2[par, i, sl2], jnp.uint32)
                fout[par, i, sl2] = (
                    lax.bitcast_convert_type(wa << 16, jnp.float32)
                    + lax.bitcast_convert_type(wb << 16, jnp.float32))
                fout[par, i, pl.ds(H2 + j * L, L)] = (
                    lax.bitcast_convert_type(
                        wa & jnp.uint32(0xFFFF0000), jnp.float32)
                    + lax.bitcast_convert_type(
                        wb & jnp.uint32(0xFFFF0000), jnp.float32))
        outd[c] = pltpu.async_copy(
            fout.at[par], out_hbm.at[pl.ds(base + c * CS, CS)], so[par])
    outd[NCH - 2].wait()
    outd[NCH - 1].wait()

  return combine


def kernel(hidden_states, gate_w, w_gate, w_up, w_down):
    b, s, h = hidden_states.shape
    x = hidden_states.reshape(-1, h)

    (e1, e2, r1, r2, w1b, w2b, offs, te, aux, xb) = _router_call(x, gate_w)

    s1, s2 = _slot_call(offs.reshape(16), e1, e2, r1, r2)
    s1 = s1.reshape(T)
    s2 = s2.reshape(T)

    gx, wslot = _make_dispatch()(xb, s1, s2, w1b, w2b)

    y = _ffn_call(te.reshape(64)[:G], gx, w_gate, w_up, w_down, wslot)

    out = _make_combine()(y, s1, s2)
    return out.reshape(b, s, h), aux.reshape(())
